# baseline (speedup 1.0000x reference)
import functools

import jax
import jax.numpy as jnp
from jax import lax
from jax.experimental import pallas as pl
from jax.experimental.pallas import tpu as pltpu
from jax.experimental.pallas import tpu_sc as plsc

_D = 64
_M = 4
_LANES = 16
_SB = 4
_BB = 32


@functools.cache
def _make_sc_call(B, S, n_opc, n_opr, interpret=False):
    try:
        info = plsc.get_sparse_core_info()
        NC, NS = info.num_cores, info.num_subcores
    except ValueError:
        NC, NS = 2, 16
    NW = NC * NS
    N = B * S
    assert B % (NW * _BB) == 0 or B == NW * _BB
    assert S % _SB == 0
    n_chunks = S // _SB
    CR = _SB * _BB
    CM = CR * _M

    mesh = plsc.VectorSubcoreMesh(
        core_axis_name="c", subcore_axis_name="s",
        num_cores=NC, num_subcores=NS)

    @functools.partial(
        pl.kernel,
        out_type=jax.ShapeDtypeStruct((N, _D), jnp.float32),
        mesh=mesh,
        interpret=interpret,
        compiler_params=pltpu.CompilerParams(use_tc_tiling_on_sc=False),
        scratch_types=[
            pltpu.VMEM((S, _BB), jnp.int32),
            pltpu.VMEM((S, _M, _BB), jnp.int32),
            pltpu.VMEM((CR,), jnp.int32),
            pltpu.VMEM((CM,), jnp.int32),
            pltpu.VMEM((CM,), jnp.int32),
            pltpu.VMEM((CM,), jnp.float32),
            pltpu.VMEM((CM, 2 * _D), jnp.float32),
            pltpu.VMEM((CR, _D), jnp.float32),
            pltpu.SemaphoreType.DMA,
            pltpu.SemaphoreType.DMA,
        ],
    )
    def sc_fn(opc_ids_hbm, opr_ids_hbm, opc_tab_hbm, opr_tab_hbm, out_hbm,
              opcb_v, idsb_v, opc_f, ids_f, h_v, w_v, rows_v, o_v,
              sem_ids, sem_g):
        wid = lax.axis_index("s") * NC + lax.axis_index("c")
        b0 = wid * _BB

        pltpu.async_copy(
            opc_ids_hbm.at[:, pl.ds(b0, _BB)], opcb_v, sem_ids).wait()
        pltpu.async_copy(
            opr_ids_hbm.at[:, :, pl.ds(b0, _BB)], idsb_v, sem_ids).wait()

        def chunk_body(k, carry):
            s0 = k * _SB
            for si in range(_SB):
                for h in range(_BB // _LANES):
                    sl = pl.ds(h * _LANES, _LANES)
                    opc_f[pl.ds(si * _BB + h * _LANES, _LANES)] = (
                        opcb_v[s0 + si, sl])
                    idv = [idsb_v[s0 + si, m, sl] for m in range(_M)]
                    mk = [jnp.where(v != 0, 1.0, 0.0) for v in idv]
                    cnt = mk[0] + mk[1] + mk[2] + mk[3] + 1e-10
                    for m in range(_M):
                        off = (si * _M + m) * _BB + h * _LANES
                        ids_f[pl.ds(off, _LANES)] = idv[m] >> 1
                        h_v[pl.ds(off, _LANES)] = (idv[m] & 1) * _D
                        w_v[pl.ds(off, _LANES)] = mk[m] / cnt
            gs = [pltpu.async_copy(opc_tab_hbm.at[opc_f], o_v, sem_g)]
            for q in range(_SB):
                gs.append(pltpu.async_copy(
                    opr_tab_hbm.at[ids_f.at[pl.ds(q * _M * _BB, _M * _BB)]],
                    rows_v.at[pl.ds(q * _M * _BB, _M * _BB)], sem_g))
            for g in gs:
                g.wait()

            def acc_body(t, carry2):
                si = t >> 1
                h = t & 1
                wvecs = [w_v[pl.ds((si * _M + m) * _BB + h * _LANES, _LANES)]
                         for m in range(_M)]
                hvecs = [h_v[pl.ds((si * _M + m) * _BB + h * _LANES, _LANES)]
                         for m in range(_M)]
                for j in range(_LANES):
                    r = si * _M * _BB + h * _LANES + j
                    o_r = si * _BB + h * _LANES + j
                    hs = [hvecs[m][j] for m in range(_M)]
                    for dblk in range(_D // _LANES):
                        acc = wvecs[0][j] * rows_v[
                            r, pl.ds(hs[0] + dblk * _LANES, _LANES)]
                        for m in range(1, _M):
                            acc = acc + wvecs[m][j] * rows_v[
                                r + m * _BB,
                                pl.ds(hs[m] + dblk * _LANES, _LANES)]
                        plsc.addupdate(
                            o_v.at[o_r, pl.ds(dblk * _LANES, _LANES)], acc)
                return carry2

            lax.fori_loop(0, _SB * (_BB // _LANES), acc_body, 0)
            ocs = [pltpu.async_copy(
                       o_v.at[pl.ds(si * _BB, _BB)],
                       out_hbm.at[pl.ds((s0 + si) * B + b0, _BB)],
                       sem_ids)
                   for si in range(_SB)]
            for oc in ocs:
                oc.wait()
            return carry

        lax.fori_loop(0, n_chunks, chunk_body, 0)

    return sc_fn


def kernel(opcode_ids, operand_ids, opcode_table, operand_table):
    B, S = opcode_ids.shape
    fn = _make_sc_call(B, S, opcode_table.shape[0], operand_table.shape[0])
    opc_t = opcode_ids.T.astype(jnp.int32)
    opr_t = jnp.transpose(operand_ids, (1, 2, 0)).astype(jnp.int32)
    opr_tab2 = lax.optimization_barrier(
        operand_table.reshape(operand_table.shape[0] // 2, 2 * _D))
    out = fn(opc_t, opr_t, opcode_table, opr_tab2)
    return jnp.transpose(out.reshape(S, B, _D), (1, 0, 2))


# speedup vs baseline: 1.1944x; 1.1944x over previous
import functools

import jax
import jax.numpy as jnp
from jax import lax
from jax.experimental import pallas as pl
from jax.experimental.pallas import tpu as pltpu
from jax.experimental.pallas import tpu_sc as plsc

_D = 64
_M = 4
_CHUNK = 128
_LANES = 16


@functools.cache
def _make_sc_call(N, n_opc, n_opr, interpret=False):
    try:
        info = plsc.get_sparse_core_info()
        NC, NS = info.num_cores, info.num_subcores
    except ValueError:
        NC, NS = 2, 16
    NW = NC * NS
    assert N % (NW * _CHUNK) == 0
    per_w = N // NW
    n_chunks = per_w // _CHUNK

    mesh = plsc.VectorSubcoreMesh(
        core_axis_name="c", subcore_axis_name="s",
        num_cores=NC, num_subcores=NS)

    @functools.partial(
        pl.kernel,
        out_type=jax.ShapeDtypeStruct((N, _D), jnp.float32),
        mesh=mesh,
        interpret=interpret,
        compiler_params=pltpu.CompilerParams(use_tc_tiling_on_sc=False),
        scratch_types=[
            pltpu.VMEM((_CHUNK,), jnp.int32),
            pltpu.VMEM((_M, _CHUNK), jnp.int32),
            pltpu.VMEM((_M, _CHUNK), jnp.float32),
            pltpu.VMEM((_M, _CHUNK, _D), jnp.float32),
            pltpu.VMEM((_CHUNK, _D), jnp.float32),
            pltpu.SemaphoreType.DMA,
            pltpu.SemaphoreType.DMA,
        ],
    )
    def sc_fn(opc_ids_hbm, opr_ids_hbm, opc_tab_hbm, opr_tab_hbm, out_hbm,
              opc_v, ids_v, w_v, rows_v, o_v, sem_ids, sem_g):
        wid = lax.axis_index("s") * NC + lax.axis_index("c")
        w_base = wid * per_w

        def chunk_body(c, carry):
            base = w_base + c * _CHUNK
            cps = [pltpu.async_copy(
                opc_ids_hbm.at[pl.ds(base, _CHUNK)], opc_v, sem_ids)]
            for m in range(_M):
                cps.append(pltpu.async_copy(
                    opr_ids_hbm.at[m, pl.ds(base, _CHUNK)], ids_v.at[m],
                    sem_ids))
            for cp in cps:
                cp.wait()
            gs = [pltpu.async_copy(opc_tab_hbm.at[opc_v], o_v, sem_g)]
            for m in range(_M):
                gs.append(pltpu.async_copy(
                    opr_tab_hbm.at[ids_v.at[m]], rows_v.at[m], sem_g))
            for t in range(_CHUNK // _LANES):
                sl = pl.ds(t * _LANES, _LANES)
                mk = [jnp.where(ids_v[m, sl] != 0, 1.0, 0.0) for m in range(_M)]
                cnt = mk[0] + mk[1] + mk[2] + mk[3] + 1e-10
                for m in range(_M):
                    w_v[m, sl] = mk[m] / cnt
            for g in gs:
                g.wait()

            def group_body(g, carry2):
                i0 = g * _LANES
                wvecs = [w_v[m, pl.ds(i0, _LANES)] for m in range(_M)]
                for j in range(_LANES):
                    i = i0 + j
                    ws = [wvecs[m][j] for m in range(_M)]
                    for dblk in range(_D // _LANES):
                        sl = pl.ds(dblk * _LANES, _LANES)
                        acc = ws[0] * rows_v[0, i, sl]
                        for m in range(1, _M):
                            acc = acc + ws[m] * rows_v[m, i, sl]
                        plsc.addupdate(o_v.at[i, sl], acc)
                return carry2

            lax.fori_loop(0, _CHUNK // _LANES, group_body, 0)
            pltpu.sync_copy(o_v, out_hbm.at[pl.ds(base, _CHUNK)])
            return carry

        lax.fori_loop(0, n_chunks, chunk_body, 0)

    return sc_fn


def kernel(opcode_ids, operand_ids, opcode_table, operand_table):
    B, S = opcode_ids.shape
    N = B * S
    opc_flat = opcode_ids.reshape(N).astype(jnp.int32)
    opr_t = operand_ids.reshape(N, _M).T.astype(jnp.int32)
    fn = _make_sc_call(N, opcode_table.shape[0], operand_table.shape[0])
    out = fn(opc_flat, opr_t, opcode_table, operand_table)
    return out.reshape(B, S, _D)
